# baseline (speedup 1.0000x reference)
import functools

import jax
import jax.numpy as jnp
from jax import lax
from jax.experimental import pallas as pl
from jax.experimental.pallas import tpu as pltpu
from jax.experimental.pallas import tpu_sc as plsc

N = 10000
E = 320000
D = 128
D_OUT = 512

NC = 2
NS = 16
NT = NC * NS
FPT = D // NT
L = 16
SEG = 2048
NSEGS = -(-E // SEG)
EP = NSEGS * SEG
DUMMY = N
NPAD = N + 8
STEPS = SEG // L
UNROLL = 16
BLOCKS = STEPS // UNROLL
XW = N * FPT
AW = NPAD * FPT


def _sc_aggregate(xt, idx_w):
    mesh = plsc.VectorSubcoreMesh(core_axis_name="c", subcore_axis_name="s")

    SEGW = 2 * SEG

    @functools.partial(
        pl.kernel,
        out_type=jax.ShapeDtypeStruct((NT * XW,), jnp.float32),
        mesh=mesh,
        scratch_types=[
            pltpu.VMEM((2, SEGW), jnp.int32),
            pltpu.VMEM((XW,), jnp.float32),
            pltpu.VMEM((AW,), jnp.float32),
            pltpu.SemaphoreType.DMA,
        ],
        compiler_params=pltpu.CompilerParams(needs_layout_passes=False),
    )
    def sc_kernel(xt_hbm, idx_hbm, out_hbm, idx_v, x_v, acc_v, isem):
        c = lax.axis_index("c")
        s = lax.axis_index("s")
        w = c * NS + s

        pltpu.sync_copy(xt_hbm.at[pl.ds(w * XW, XW)], x_v)
        pltpu.sync_copy(xt_hbm.at[pl.ds(w * XW, XW)], acc_v.at[pl.ds(0, XW)])
        for i in range((AW - XW) // L):
            acc_v[pl.ds(XW + i * L, L)] = jnp.zeros((L,), jnp.float32)

        pltpu.async_copy(idx_hbm.at[pl.ds(0, SEGW)], idx_v.at[0], isem)

        def step(p, k):
            off = k * L
            row = idx_v[p, pl.ds(off, L)]
            col = idx_v[p, pl.ds(SEG + off, L)]
            colb = col << 2
            rowb = row << 2
            for f in range(FPT):
                g = plsc.load_gather(x_v, (colb + f,))
                plsc.addupdate_scatter(acc_v, (rowb + f,), g)

        def seg_body(sg, carry):
            p = lax.rem(sg, 2)
            pltpu.make_async_copy(idx_hbm.at[pl.ds(0, SEGW)], idx_v.at[p],
                                  isem).wait()

            @pl.when(sg + 1 < NSEGS)
            def _():
                pltpu.async_copy(idx_hbm.at[pl.ds((sg + 1) * SEGW, SEGW)],
                                 idx_v.at[1 - p], isem)

            def block(bk, carry2):
                base = bk * UNROLL
                for u in range(UNROLL):
                    step(p, base + u)
                return carry2

            lax.fori_loop(0, BLOCKS, block, 0)
            return carry

        lax.fori_loop(0, NSEGS, seg_body, 0)

        pltpu.sync_copy(acc_v.at[pl.ds(0, XW)],
                        out_hbm.at[pl.ds(w * XW, XW)])

    return sc_kernel(xt, idx_w)


def _combine_matmul(acc, W, b):
    BLK = 1000
    grid = N // BLK

    def tc_kernel(a_ref, w_ref, b_ref, o_ref):
        o_ref[...] = lax.dot_general(
            a_ref[...], w_ref[...], (((1,), (1,)), ((), ())),
            preferred_element_type=jnp.float32) + b_ref[...]

    return pl.pallas_call(
        tc_kernel,
        grid=(grid,),
        in_specs=[
            pl.BlockSpec((BLK, D), lambda i: (i, 0)),
            pl.BlockSpec((D_OUT, D), lambda i: (0, 0)),
            pl.BlockSpec((1, D_OUT), lambda i: (0, 0)),
        ],
        out_specs=pl.BlockSpec((BLK, D_OUT), lambda i: (i, 0)),
        out_shape=jax.ShapeDtypeStruct((N, D_OUT), jnp.float32),
    )(acc, W, b.reshape(1, D_OUT))


def kernel(x, edge_index, W, b):
    ei = edge_index.astype(jnp.int32)
    row = ei[0]
    col = ei[1]
    pad = EP - E
    row_p = jnp.concatenate([row, jnp.full((pad,), DUMMY, jnp.int32)])
    col_p = jnp.concatenate([col, jnp.zeros((pad,), jnp.int32)])
    idx_w = jnp.stack([row_p.reshape(NSEGS, SEG),
                       col_p.reshape(NSEGS, SEG)],
                      axis=1).reshape(NSEGS * 2 * SEG)
    xt = x.reshape(N, NT, FPT).transpose(1, 0, 2).reshape(NT * XW)
    acc_t = _sc_aggregate(xt, idx_w)
    acc = acc_t.reshape(NT, N, FPT).transpose(1, 0, 2).reshape(N, D)
    return _combine_matmul(acc, W, b)


# speedup vs baseline: 2.1275x; 2.1275x over previous
import functools

import jax
import jax.numpy as jnp
from jax import lax
from jax.experimental import pallas as pl
from jax.experimental.pallas import tpu as pltpu
from jax.experimental.pallas import tpu_sc as plsc

N = 10000
E = 320000
D = 128
D_OUT = 512

NC = 2
NS = 16
NW = NC * NS
CHUNK = 128
NBUF = 2
NCHUNKH = 40
NCHUNK = 2 * NCHUNKH
EPW = NCHUNK * CHUNK
EP = NW * EPW
DUMMY = N
NPAD = N + 8
RPT = 632
RPT_LAST = N - (NS - 1) * RPT


def _sc_aggregate(x, col_w, row_w):
    mesh = plsc.VectorSubcoreMesh(core_axis_name="c", subcore_axis_name="s")

    @functools.partial(
        pl.kernel,
        out_type=jax.ShapeDtypeStruct((NC, N, D), jnp.float32),
        mesh=mesh,
        scratch_types=[
            pltpu.VMEM((NCHUNK, CHUNK), jnp.int32),
            pltpu.VMEM((NCHUNKH, CHUNK), jnp.int32),
            pltpu.VMEM((NBUF, CHUNK, D), jnp.float32),
            pltpu.VMEM_SHARED((NPAD, D), jnp.float32),
            pltpu.SemaphoreType.DMA((NBUF,)),
        ],
    )
    def sc_kernel(x_hbm, col_hbm, row_hbm, out_hbm, col_v, row_v, rows_v,
                  acc_sh, gsem):
        c = lax.axis_index("c")
        s = lax.axis_index("s")
        wid = c * NS + s

        pltpu.sync_copy(col_hbm.at[wid], col_v)
        pltpu.sync_copy(row_hbm.at[wid, 0], row_v)

        @pl.when(s < NS - 1)
        def _():
            pltpu.sync_copy(x_hbm.at[pl.ds(s * RPT, RPT)],
                            acc_sh.at[pl.ds(s * RPT, RPT)])

        @pl.when(s == NS - 1)
        def _():
            pltpu.sync_copy(x_hbm.at[pl.ds((NS - 1) * RPT, RPT_LAST)],
                            acc_sh.at[pl.ds((NS - 1) * RPT, RPT_LAST)])

        plsc.subcore_barrier()

        def start_gather(j, b):
            pltpu.async_copy(x_hbm.at[col_v.at[j]], rows_v.at[b], gsem.at[b])

        def wait_gather(j, b):
            pltpu.make_async_copy(x_hbm.at[col_v.at[j]], rows_v.at[b],
                                  gsem.at[b]).wait()

        for b in range(NBUF):
            start_gather(b, b)

        def body(j, carry):
            b = lax.rem(j, NBUF)
            wait_gather(j, b)

            @pl.when(j == NCHUNKH)
            def _():
                pltpu.sync_copy(row_hbm.at[wid, 1], row_v)

            pltpu.sync_copy(rows_v.at[b], acc_sh.at[row_v.at[lax.rem(j, NCHUNKH)]],
                            add=True)

            @pl.when(j + NBUF < NCHUNK)
            def _():
                start_gather(j + NBUF, b)

            return carry

        lax.fori_loop(0, NCHUNK, body, 0)
        plsc.subcore_barrier()

        @pl.when(s < NS - 1)
        def _():
            pltpu.sync_copy(acc_sh.at[pl.ds(s * RPT, RPT)],
                            out_hbm.at[c, pl.ds(s * RPT, RPT)])

        @pl.when(s == NS - 1)
        def _():
            pltpu.sync_copy(acc_sh.at[pl.ds((NS - 1) * RPT, RPT_LAST)],
                            out_hbm.at[c, pl.ds((NS - 1) * RPT, RPT_LAST)])

    return sc_kernel(x, col_w, row_w)


def _combine_matmul(x, acc, W, b):
    BLK = 1000
    grid = N // BLK

    def tc_kernel(x_ref, a0_ref, a1_ref, w_ref, b_ref, o_ref):
        sm = a0_ref[...] + a1_ref[...] - x_ref[...]
        o_ref[...] = lax.dot_general(
            sm, w_ref[...], (((1,), (1,)), ((), ())),
            preferred_element_type=jnp.float32) + b_ref[...]

    return pl.pallas_call(
        tc_kernel,
        grid=(grid,),
        in_specs=[
            pl.BlockSpec((BLK, D), lambda i: (i, 0)),
            pl.BlockSpec((BLK, D), lambda i: (i, 0)),
            pl.BlockSpec((BLK, D), lambda i: (i, 0)),
            pl.BlockSpec((D_OUT, D), lambda i: (0, 0)),
            pl.BlockSpec((1, D_OUT), lambda i: (0, 0)),
        ],
        out_specs=pl.BlockSpec((BLK, D_OUT), lambda i: (i, 0)),
        out_shape=jax.ShapeDtypeStruct((N, D_OUT), jnp.float32),
    )(x, acc[0], acc[1], W, b.reshape(1, D_OUT))


def kernel(x, edge_index, W, b):
    ei = edge_index.astype(jnp.int32)
    row = ei[0]
    col = ei[1]
    pad = EP - E
    row_w = jnp.concatenate(
        [row, jnp.full((pad,), DUMMY, jnp.int32)]).reshape(
            NW, 2, NCHUNKH, CHUNK)
    col_w = jnp.concatenate(
        [col, jnp.zeros((pad,), jnp.int32)]).reshape(NW, NCHUNK, CHUNK)
    acc = _sc_aggregate(x, col_w, row_w)
    return _combine_matmul(x, acc, W, b)
